# baseline (speedup 1.0000x reference)
import functools

import jax
import jax.numpy as jnp
from jax import lax
from jax.experimental import pallas as pl
from jax.experimental.pallas import tpu as pltpu
from jax.experimental.pallas import tpu_sc as plsc

_NC = 2
_NS = 16
_NW = _NC * _NS
_CHUNK = 128
_K = 10


@functools.lru_cache(maxsize=None)
def _build(n_total, vocab, dim):
    b_per_w = n_total // _NW
    n_chunks = b_per_w // _CHUNK
    n_outer = -(-n_chunks // _K) * _K
    mesh = plsc.VectorSubcoreMesh(core_axis_name="c", subcore_axis_name="s")

    def body(table_hbm, idx_hbm, out_hbm, idx_v, rows_v, gsems, osems):
        cid = lax.axis_index("c")
        sid = lax.axis_index("s")
        wid = sid * _NC + cid
        pltpu.sync_copy(idx_hbm.at[wid], idx_v)
        row_base = wid * b_per_w

        def fire_gather(j, s):
            pltpu.async_copy(
                table_hbm.at[idx_v.at[j]],
                rows_v.at[s],
                gsems[s],
            )

        def wait_gather(s):
            pltpu.make_async_copy(
                table_hbm.at[idx_v.at[0]],
                rows_v.at[s],
                gsems[s],
            ).wait()

        def wait_write(s):
            pltpu.make_async_copy(
                table_hbm.at[pl.ds(0, _CHUNK)],
                rows_v.at[s],
                osems[s],
            ).wait()

        for s in range(_K - 1):
            fire_gather(s, s)

        @pl.loop(0, n_outer, step=_K)
        def outer(j0):
            for s in range(_K):
                j = j0 + s

                @pl.when(j < n_chunks)
                def _():
                    wait_gather(s)
                    pltpu.async_copy(
                        rows_v.at[s],
                        out_hbm.at[pl.ds(row_base + j * _CHUNK, _CHUNK)],
                        osems[s],
                    )

                jf = j + _K - 1
                sf = (s + _K - 1) % _K

                @pl.when(jf < n_chunks)
                def _():
                    @pl.when(j >= 1)
                    def _():
                        wait_write(sf)

                    fire_gather(jf, sf)

        for j in range(n_chunks - _K, n_chunks):
            wait_write(j % _K)

    kern = pl.kernel(
        body,
        out_type=jax.ShapeDtypeStruct((n_total, dim), jnp.float32),
        mesh=mesh,
        scratch_types=[
            pltpu.VMEM((n_chunks, _CHUNK), jnp.int32),
            pltpu.VMEM((_K, _CHUNK, dim), jnp.float32),
            [pltpu.SemaphoreType.DMA] * _K,
            [pltpu.SemaphoreType.DMA] * _K,
        ],
        compiler_params=pltpu.CompilerParams(use_tc_tiling_on_sc=False),
    )
    return kern


def kernel(x, table):
    b, l = x.shape
    vocab, dim = table.shape
    n_total = b * l
    idx = x.reshape(_NW, n_total // (_NW * _CHUNK), _CHUNK).astype(jnp.int32)
    out = _build(n_total, vocab, dim)(table, idx)
    return out.reshape(b, l, dim)


# speedup vs baseline: 1.0003x; 1.0003x over previous
import functools

import jax
import jax.numpy as jnp
from jax import lax
from jax.experimental import pallas as pl
from jax.experimental.pallas import tpu as pltpu
from jax.experimental.pallas import tpu_sc as plsc

_NC = 2
_NS = 16
_NW = _NC * _NS
_K = 8


@functools.lru_cache(maxsize=None)
def _build(batch, seq, vocab, dim):
    rows_per_w = batch // _NW
    mesh = plsc.VectorSubcoreMesh(core_axis_name="c", subcore_axis_name="s")

    def body(x_hbm, table_hbm, out_hbm, idx_v, rows_v, gsems, osems):
        cid = lax.axis_index("c")
        sid = lax.axis_index("s")
        wid = sid * _NC + cid
        base = wid * rows_per_w
        pltpu.sync_copy(x_hbm.at[pl.ds(base, rows_per_w)], idx_v)

        def fire_gather(j, s):
            pltpu.async_copy(
                table_hbm.at[idx_v.at[j]],
                rows_v.at[s],
                gsems[s],
            )

        def wait_gather(s):
            pltpu.make_async_copy(
                table_hbm.at[idx_v.at[0]],
                rows_v.at[s],
                gsems[s],
            ).wait()

        def wait_write(s):
            pltpu.make_async_copy(
                table_hbm.at[pl.ds(0, seq)],
                rows_v.at[s],
                osems[s],
            ).wait()

        for s in range(_K - 1):
            fire_gather(s, s)

        @pl.loop(0, rows_per_w, step=_K)
        def outer(j0):
            for s in range(_K):
                j = j0 + s
                wait_gather(s)
                pltpu.async_copy(
                    rows_v.at[s],
                    out_hbm.at[base + j],
                    osems[s],
                )

                jf = j + _K - 1
                sf = (s + _K - 1) % _K

                @pl.when(jf < rows_per_w)
                def _():
                    @pl.when(j >= 1)
                    def _():
                        wait_write(sf)

                    fire_gather(jf, sf)

        for j in range(rows_per_w - _K, rows_per_w):
            wait_write(j % _K)

    kern = pl.kernel(
        body,
        out_type=jax.ShapeDtypeStruct((batch, seq, dim), jnp.float32),
        mesh=mesh,
        scratch_types=[
            pltpu.VMEM((rows_per_w, seq), jnp.int32),
            pltpu.VMEM((_K, seq, dim), jnp.float32),
            [pltpu.SemaphoreType.DMA] * _K,
            [pltpu.SemaphoreType.DMA] * _K,
        ],
        compiler_params=pltpu.CompilerParams(use_tc_tiling_on_sc=False),
    )
    return kern


def kernel(x, table):
    b, l = x.shape
    vocab, dim = table.shape
    return _build(b, l, vocab, dim)(x, table)


# speedup vs baseline: 1.0044x; 1.0042x over previous
import functools

import jax
import jax.numpy as jnp
from jax import lax
from jax.experimental import pallas as pl
from jax.experimental.pallas import tpu as pltpu
from jax.experimental.pallas import tpu_sc as plsc

_NC = 2
_NS = 16
_NW = _NC * _NS
_K = 4


@functools.lru_cache(maxsize=None)
def _build(batch, seq, vocab, dim):
    rows_per_w = batch // _NW
    mesh = plsc.VectorSubcoreMesh(core_axis_name="c", subcore_axis_name="s")

    def body(x_hbm, table_hbm, out_hbm, idx_v, rows_v, rows_sh, gsems, osems):
        cid = lax.axis_index("c")
        sid = lax.axis_index("s")
        wid = sid * _NC + cid
        base = wid * rows_per_w
        spm = rows_sh.at[sid]
        pltpu.sync_copy(x_hbm.at[pl.ds(base, rows_per_w)], idx_v)

        def fire_gather(j, s):
            pltpu.async_copy(
                table_hbm.at[idx_v.at[j]],
                rows_v.at[s],
                gsems[s],
            )

        def wait_gather(s):
            pltpu.make_async_copy(
                table_hbm.at[idx_v.at[0]],
                rows_v.at[s],
                gsems[s],
            ).wait()

        def wait_write(s):
            pltpu.make_async_copy(
                spm.at[s],
                out_hbm.at[base],
                osems[s],
            ).wait()

        for s in range(_K - 1):
            fire_gather(s, s)

        @pl.loop(0, rows_per_w, step=_K)
        def outer(j0):
            for s in range(_K):
                j = j0 + s
                wait_gather(s)

                @pl.when(j >= _K)
                def _():
                    wait_write(s)

                pltpu.sync_copy(rows_v.at[s], spm.at[s])
                pltpu.async_copy(
                    spm.at[s],
                    out_hbm.at[base + j],
                    osems[s],
                )

                jf = j + _K - 1
                sf = (s + _K - 1) % _K

                @pl.when(jf < rows_per_w)
                def _():
                    fire_gather(jf, sf)

        for s in range(_K):
            wait_write(s)

    kern = pl.kernel(
        body,
        out_type=jax.ShapeDtypeStruct((batch, seq, dim), jnp.float32),
        mesh=mesh,
        scratch_types=[
            pltpu.VMEM((rows_per_w, seq), jnp.int32),
            pltpu.VMEM((_K, seq, dim), jnp.float32),
            pltpu.VMEM_SHARED((_NS, _K, seq, dim), jnp.float32),
            [pltpu.SemaphoreType.DMA] * _K,
            [pltpu.SemaphoreType.DMA] * _K,
        ],
        compiler_params=pltpu.CompilerParams(use_tc_tiling_on_sc=False),
    )
    return kern


def kernel(x, table):
    b, l = x.shape
    vocab, dim = table.shape
    return _build(b, l, vocab, dim)(x, table)


# speedup vs baseline: 1.2708x; 1.2651x over previous
import functools

import jax
import jax.numpy as jnp
from jax import lax
from jax.experimental import pallas as pl
from jax.experimental.pallas import tpu as pltpu
from jax.experimental.pallas import tpu_sc as plsc

_NC = 2
_NS = 16
_NW = _NC * _NS
_K = 4


@functools.lru_cache(maxsize=None)
def _build(batch, seq, vocab, dim):
    rows_per_w = batch // _NW
    n_idx = rows_per_w * seq
    mesh = plsc.VectorSubcoreMesh(core_axis_name="c", subcore_axis_name="s")

    def body(x_hbm, table_hbm, out_hbm, idx_v, rows_v, gsems, osems):
        cid = lax.axis_index("c")
        sid = lax.axis_index("s")
        wid = sid * _NC + cid
        base = wid * rows_per_w
        pltpu.sync_copy(x_hbm.at[pl.ds(base * seq, n_idx)], idx_v)

        def fire_gather(j, s):
            pltpu.async_copy(
                table_hbm.at[idx_v.at[pl.ds(j * seq, seq)]],
                rows_v.at[s],
                gsems[s],
            )

        def wait_gather(s):
            pltpu.make_async_copy(
                table_hbm.at[idx_v.at[pl.ds(0, seq)]],
                rows_v.at[s],
                gsems[s],
            ).wait()

        def fire_write(j, s):
            pltpu.async_copy(
                rows_v.at[s],
                out_hbm.at[base + j],
                osems[s],
            )

        def wait_write(s):
            pltpu.make_async_copy(
                rows_v.at[s],
                out_hbm.at[base],
                osems[s],
            ).wait()

        for s in range(_K - 1):
            fire_gather(s, s)

        @pl.loop(0, rows_per_w, step=_K)
        def outer(j0):
            for s in range(_K):
                j = j0 + s
                wait_gather(s)
                fire_write(j, s)

                jf = j + _K - 1
                sf = (s + _K - 1) % _K

                @pl.when(jf < rows_per_w)
                def _():
                    @pl.when(j >= 1)
                    def _():
                        wait_write(sf)

                    fire_gather(jf, sf)

        for j in range(rows_per_w - _K, rows_per_w):
            wait_write(j % _K)

    kern = pl.kernel(
        body,
        out_type=jax.ShapeDtypeStruct((batch, seq, 2 * dim), jnp.float32),
        mesh=mesh,
        scratch_types=[
            pltpu.VMEM((n_idx,), jnp.int32),
            pltpu.VMEM((_K, seq, 2 * dim), jnp.float32),
            [pltpu.SemaphoreType.DMA] * _K,
            [pltpu.SemaphoreType.DMA] * _K,
        ],
    )
    return kern


def kernel(x, table):
    b, l = x.shape
    vocab, dim = table.shape
    table_w = jnp.concatenate([table, table], axis=1)
    out_w = _build(b, l, vocab, dim)(x.reshape(b * l), table_w)
    return out_w[:, :, :dim]
